# Initial kernel scaffold; baseline (speedup 1.0000x reference)
import jax
import jax.numpy as jnp
from jax.experimental import pallas as pl


def kernel(feat_stats, alt_idx, alt_embed, W1, b1, Wg1, bg1, Wg2, bg2):
    raise NotImplementedError("write your pallas kernel here")


# speedup vs baseline: 1.7364x; 1.7364x over previous
import functools

import jax
import jax.numpy as jnp
from jax import lax
from jax.experimental import pallas as pl


def _router_body(num_alt, k_top, x_ref, alt_ref, altemb_ref, w1_ref, b1_ref,
                 wg1a_ref, wg1b_ref, bg1_ref, wg2_ref, bg2_ref,
                 gw_ref, idx_ref):
    x = x_ref[...]
    f = jnp.maximum(
        jnp.dot(x, w1_ref[...], preferred_element_type=jnp.float32)
        + b1_ref[...], 0.0)

    t2 = jnp.dot(altemb_ref[...], wg1b_ref[...],
                 preferred_element_type=jnp.float32)
    aid = alt_ref[...]
    acc = jnp.dot(f, wg1a_ref[...],
                  preferred_element_type=jnp.float32) + bg1_ref[...]
    for j in range(num_alt):
        acc = acc + jnp.where(aid == j, 1.0, 0.0) * t2[j:j + 1, :]
    h = jnp.maximum(acc, 0.0)

    logits = jnp.dot(h, wg2_ref[...],
                     preferred_element_type=jnp.float32) + bg2_ref[...]
    m = jnp.max(logits, axis=1, keepdims=True)
    e = jnp.exp(logits - m)
    s = jnp.sum(e, axis=1, keepdims=True)
    gw = e / s
    gw_ref[...] = gw

    blk, E = gw.shape
    iota = lax.broadcasted_iota(jnp.int32, (blk, E), 1)
    work = gw
    for k in range(k_top):
        mx = jnp.max(work, axis=1, keepdims=True)
        cand = jnp.where(work == mx, iota, E)
        sel = jnp.min(cand, axis=1, keepdims=True)
        idx_ref[:, k:k + 1] = sel
        work = jnp.where(iota == sel, -1.0, work)


def kernel(feat_stats, alt_idx, alt_embed, W1, b1, Wg1, bg1, Wg2, bg2):
    B, D = feat_stats.shape
    num_alt, H = alt_embed.shape
    E = Wg2.shape[1]
    K = 8
    blk = 2048

    Wg1a = Wg1[:H]
    Wg1b = Wg1[H:]
    alt_pad = jnp.zeros((8, H), jnp.float32).at[:num_alt].set(alt_embed)
    alt2d = alt_idx.astype(jnp.int32).reshape(B, 1)
    b1r = b1.reshape(1, H)
    bg1r = bg1.reshape(1, H)
    bg2r = bg2.reshape(1, E)

    grid = (B // blk,)
    row = lambda i: (i, 0)
    rep = lambda i: (0, 0)
    gw, idx = pl.pallas_call(
        functools.partial(_router_body, num_alt, K),
        grid=grid,
        in_specs=[
            pl.BlockSpec((blk, D), row),
            pl.BlockSpec((blk, 1), row),
            pl.BlockSpec((8, H), rep),
            pl.BlockSpec((D, H), rep),
            pl.BlockSpec((1, H), rep),
            pl.BlockSpec((H, H), rep),
            pl.BlockSpec((H, H), rep),
            pl.BlockSpec((1, H), rep),
            pl.BlockSpec((H, E), rep),
            pl.BlockSpec((1, E), rep),
        ],
        out_specs=[
            pl.BlockSpec((blk, E), row),
            pl.BlockSpec((blk, K), row),
        ],
        out_shape=[
            jax.ShapeDtypeStruct((B, E), jnp.float32),
            jax.ShapeDtypeStruct((B, K), jnp.int32),
        ],
    )(feat_stats, alt2d, alt_pad, W1, b1r, Wg1a, Wg1b, bg1r, Wg2, bg2r)
    return gw, idx


# speedup vs baseline: 3.7899x; 2.1826x over previous
import functools

import jax
import jax.numpy as jnp
from jax import lax
from jax.experimental import pallas as pl


def _router_body(num_alt, k_top, x_ref, alt_ref, altemb_ref, w1_ref, b1_ref,
                 wg1a_ref, wg1b_ref, bg1_ref, wg2_ref, bg2c_ref,
                 gw_ref, idx_ref):
    x = x_ref[...]
    blk = x.shape[0]
    f = jnp.maximum(
        jnp.dot(x, w1_ref[...], preferred_element_type=jnp.float32)
        + b1_ref[...], 0.0)

    t2 = jnp.dot(altemb_ref[...], wg1b_ref[...],
                 preferred_element_type=jnp.float32)
    aid = alt_ref[...]
    oh = (aid == lax.broadcasted_iota(jnp.int32, (blk, 8), 1)
          ).astype(jnp.float32)
    acc = (jnp.dot(f, wg1a_ref[...], preferred_element_type=jnp.float32)
           + jnp.dot(oh, t2, preferred_element_type=jnp.float32)
           + bg1_ref[...])
    h = jnp.maximum(acc, 0.0)

    logits_t = lax.dot_general(
        wg2_ref[...], h, (((0,), (1,)), ((), ())),
        preferred_element_type=jnp.float32) + bg2c_ref[...]

    mx = jnp.max(logits_t, axis=0, keepdims=True)
    e = jnp.exp(logits_t - mx)
    s = jnp.sum(e, axis=0, keepdims=True)
    gw_t = e * (1.0 / s)
    gw_ref[...] = gw_t.T

    E = gw_t.shape[0]
    iota = lax.broadcasted_iota(jnp.int32, (E, blk), 0)
    work = gw_t
    rows = []
    for _ in range(k_top):
        mxk = jnp.max(work, axis=0, keepdims=True)
        cand = jnp.where(work == mxk, iota, E)
        sel = jnp.min(cand, axis=0, keepdims=True)
        rows.append(sel)
        work = jnp.where(cand == sel, -1.0, work)
    idx_t = jnp.concatenate(rows, axis=0)
    idx_ref[...] = idx_t.T


def kernel(feat_stats, alt_idx, alt_embed, W1, b1, Wg1, bg1, Wg2, bg2):
    B, D = feat_stats.shape
    num_alt, H = alt_embed.shape
    E = Wg2.shape[1]
    K = 8
    blk = 2048

    Wg1a = Wg1[:H]
    Wg1b = Wg1[H:]
    alt_pad = jnp.zeros((8, H), jnp.float32).at[:num_alt].set(alt_embed)
    alt2d = alt_idx.astype(jnp.int32).reshape(B, 1)
    b1r = b1.reshape(1, H)
    bg1r = bg1.reshape(1, H)
    bg2c = bg2.reshape(E, 1)

    grid = (B // blk,)
    row = lambda i: (i, 0)
    rep = lambda i: (0, 0)
    gw, idx = pl.pallas_call(
        functools.partial(_router_body, num_alt, K),
        grid=grid,
        in_specs=[
            pl.BlockSpec((blk, D), row),
            pl.BlockSpec((blk, 1), row),
            pl.BlockSpec((8, H), rep),
            pl.BlockSpec((D, H), rep),
            pl.BlockSpec((1, H), rep),
            pl.BlockSpec((H, H), rep),
            pl.BlockSpec((H, H), rep),
            pl.BlockSpec((1, H), rep),
            pl.BlockSpec((H, E), rep),
            pl.BlockSpec((E, 1), rep),
        ],
        out_specs=[
            pl.BlockSpec((blk, E), row),
            pl.BlockSpec((blk, K), row),
        ],
        out_shape=[
            jax.ShapeDtypeStruct((B, E), jnp.float32),
            jax.ShapeDtypeStruct((B, K), jnp.int32),
        ],
    )(feat_stats, alt2d, alt_pad, W1, b1r, Wg1a, Wg1b, bg1r, Wg2, bg2c)
    return gw, idx
